# baseline (speedup 1.0000x reference)
import dataclasses
import functools

import jax
import jax.numpy as jnp
from jax import lax
from jax.experimental import pallas as pl
from jax.experimental.pallas import tpu as pltpu
from jax.experimental.pallas import tpu_sc as plsc

N_NODES_C = 100000
N_GRAPHS_C = 256
D_C = 128
BLOCK_R = 10000
N_BLOCKS = N_NODES_C // BLOCK_R

NW = 32
IDS_PAD = 100352
PER_W = IDS_PAD // NW
PAD_CNT = IDS_PAD - N_NODES_C

_sc_mesh = plsc.VectorSubcoreMesh(core_axis_name="c", subcore_axis_name="s")

_sc_cp = pltpu.CompilerParams()
if "needs_layout_passes" in pltpu.CompilerParams.__dataclass_fields__:
    _sc_cp = dataclasses.replace(_sc_cp, needs_layout_passes=False)


@functools.partial(
    pl.kernel,
    mesh=_sc_mesh,
    compiler_params=_sc_cp,
    out_type=jax.ShapeDtypeStruct((NW, N_GRAPHS_C), jnp.float32),
    scratch_types=[
        pltpu.VMEM((PER_W,), jnp.int32),
        pltpu.VMEM((16, N_GRAPHS_C), jnp.float32),
        pltpu.VMEM((N_GRAPHS_C,), jnp.float32),
    ],
)
def _sc_counts(ids_hbm, out_hbm, ids_v, acc_v, red_v):
    wid = lax.axis_index("s") * 2 + lax.axis_index("c")
    pltpu.sync_copy(ids_hbm.at[pl.ds(wid * PER_W, PER_W)], ids_v)

    zeros16 = jnp.zeros((16,), jnp.float32)

    @pl.loop(0, N_GRAPHS_C, step=16)
    def _(c):
        for r in range(16):
            acc_v[r, pl.ds(c, 16)] = zeros16

    ones16 = jnp.ones((16,), jnp.float32)
    lanes = lax.iota(jnp.int32, 16)

    @pl.loop(0, PER_W, step=16)
    def _(j):
        idx = ids_v[pl.ds(j, 16)]
        plsc.addupdate_scatter(acc_v, [lanes, idx], ones16)

    @pl.loop(0, N_GRAPHS_C, step=16)
    def _(c):
        s = acc_v[0, pl.ds(c, 16)]
        for r in range(1, 16):
            s = s + acc_v[r, pl.ds(c, 16)]
        red_v[pl.ds(c, 16)] = s

    pltpu.sync_copy(red_v, out_hbm.at[wid])


def _tc_sums_kernel(ids_ref, x_ref, sums_ref):
    i = pl.program_id(0)

    @pl.when(i == 0)
    def _():
        sums_ref[...] = jnp.zeros_like(sums_ref)

    ids = ids_ref[0, :, :].astype(jnp.int16)
    seg = lax.broadcasted_iota(jnp.int16, (N_GRAPHS_C, BLOCK_R), 0)
    onehot = jnp.where(ids == seg, jnp.bfloat16(1.0), jnp.bfloat16(0.0))
    xb = x_ref[...].astype(jnp.bfloat16)
    sums_ref[...] += jax.lax.dot_general(
        onehot, xb, (((1,), (0,)), ((), ())),
        preferred_element_type=jnp.float32)


def _tc_decode_kernel(sums_ref, cnt_ref, u_ref, w1a_ref, w1b_ref, b1_ref,
                      w2_ref, b2_ref, out_ref):
    counts = jnp.sum(cnt_ref[...], axis=0, keepdims=True)
    pad_adj = jnp.where(
        lax.broadcasted_iota(jnp.int32, (1, N_GRAPHS_C), 1) == N_GRAPHS_C - 1,
        jnp.float32(PAD_CNT), jnp.float32(0.0))
    inv = 1.0 / jnp.maximum(counts - pad_adj, 1.0)
    r = lax.broadcasted_iota(jnp.int16, (N_GRAPHS_C, N_GRAPHS_C), 0)
    c = lax.broadcasted_iota(jnp.int16, (N_GRAPHS_C, N_GRAPHS_C), 1)
    ident = jnp.where(r == c, jnp.bfloat16(1.0), jnp.bfloat16(0.0))
    invb = jnp.broadcast_to(inv.astype(jnp.bfloat16), (8, N_GRAPHS_C))
    inv_col = jax.lax.dot_general(
        ident, invb, (((1,), (1,)), ((), ())),
        preferred_element_type=jnp.float32)
    mean = sums_ref[...] * inv_col[:, 0:1]
    u1 = jax.lax.dot_general(
        mean, w1a_ref[...], (((1,), (0,)), ((), ())),
        preferred_element_type=jnp.float32)
    u1 += jax.lax.dot_general(
        u_ref[...], w1b_ref[...], (((1,), (0,)), ((), ())),
        preferred_element_type=jnp.float32)
    u1 = jnp.maximum(u1 + b1_ref[...], 0.0)
    out = jax.lax.dot_general(
        u1, w2_ref[...], (((1,), (0,)), ((), ())),
        preferred_element_type=jnp.float32)
    out_ref[...] = out + b2_ref[...]


@jax.jit
def kernel(x, u, graph_ids, W1, b1, W2, b2):
    ids32 = graph_ids.astype(jnp.int32)
    ids_blocks = ids32.reshape(N_BLOCKS, 1, BLOCK_R)
    ids_pad = jnp.concatenate(
        [ids32, jnp.full((PAD_CNT,), N_GRAPHS_C - 1, jnp.int32)])

    sums = pl.pallas_call(
        _tc_sums_kernel,
        grid=(N_BLOCKS,),
        in_specs=[
            pl.BlockSpec((1, 1, BLOCK_R), lambda i: (i, 0, 0)),
            pl.BlockSpec((BLOCK_R, D_C), lambda i: (i, 0)),
        ],
        out_specs=pl.BlockSpec((N_GRAPHS_C, D_C), lambda i: (0, 0)),
        out_shape=jax.ShapeDtypeStruct((N_GRAPHS_C, D_C), jnp.float32),
    )(ids_blocks, x)

    cnt_parts = _sc_counts(ids_pad)

    rep = lambda shape: pl.BlockSpec(shape, lambda: (0,) * len(shape))
    out = pl.pallas_call(
        _tc_decode_kernel,
        in_specs=[
            rep((N_GRAPHS_C, D_C)),
            rep((NW, N_GRAPHS_C)),
            rep((N_GRAPHS_C, D_C)),
            rep((D_C, D_C)),
            rep((D_C, D_C)),
            rep((1, D_C)),
            rep((D_C, D_C)),
            rep((1, D_C)),
        ],
        out_specs=rep((N_GRAPHS_C, D_C)),
        out_shape=jax.ShapeDtypeStruct((N_GRAPHS_C, D_C), jnp.float32),
    )(sums, cnt_parts, u, W1[:D_C], W1[D_C:], b1.reshape(1, D_C), W2,
      b2.reshape(1, D_C))
    return out


# speedup vs baseline: 1.2078x; 1.2078x over previous
import jax
import jax.numpy as jnp
from jax.experimental import pallas as pl
from jax.experimental.pallas import tpu as pltpu

N_NODES_C = 100000
N_GRAPHS_C = 256
D_C = 128
BLOCK_R = 20000
N_BLOCKS = N_NODES_C // BLOCK_R


def _gnavg_kernel(ids_ref, x_ref, u_ref, w1a_ref, w1b_ref, b1_ref, w2_ref,
                  b2_ref, out_ref, sums_ref, cnts_ref):
    i = pl.program_id(0)

    @pl.when(i == 0)
    def _():
        sums_ref[...] = jnp.zeros_like(sums_ref)
        cnts_ref[...] = jnp.zeros_like(cnts_ref)

    ids = ids_ref[0, :, :].astype(jnp.int16)
    seg = jax.lax.broadcasted_iota(jnp.int16, (N_GRAPHS_C, BLOCK_R), 0)
    onehot = jnp.where(ids == seg, jnp.bfloat16(1.0), jnp.bfloat16(0.0))
    xb = x_ref[...].astype(jnp.bfloat16)
    psum = jax.lax.dot_general(
        onehot, xb, (((1,), (0,)), ((), ())),
        preferred_element_type=jnp.float32)
    sums_ref[...] += psum
    ones = jnp.ones((BLOCK_R, 8), jnp.bfloat16)
    pcnt = jax.lax.dot_general(
        onehot, ones, (((1,), (0,)), ((), ())),
        preferred_element_type=jnp.float32)
    cnts_ref[...] += pcnt

    @pl.when(i == N_BLOCKS - 1)
    def _():
        counts = cnts_ref[:, 0:1]
        mean = sums_ref[...] / jnp.maximum(counts, 1.0)
        u1 = jax.lax.dot_general(
            mean, w1a_ref[...], (((1,), (0,)), ((), ())),
            preferred_element_type=jnp.float32)
        u1 += jax.lax.dot_general(
            u_ref[...], w1b_ref[...], (((1,), (0,)), ((), ())),
            preferred_element_type=jnp.float32)
        u1 = jnp.maximum(u1 + b1_ref[...], 0.0)
        out = jax.lax.dot_general(
            u1, w2_ref[...], (((1,), (0,)), ((), ())),
            preferred_element_type=jnp.float32)
        out_ref[...] = out + b2_ref[...]


@jax.jit
def kernel(x, u, graph_ids, W1, b1, W2, b2):
    ids = graph_ids.astype(jnp.int32).reshape(N_BLOCKS, 1, BLOCK_R)
    w1a = W1[:D_C]
    w1b = W1[D_C:]
    b1r = b1.reshape(1, D_C)
    b2r = b2.reshape(1, D_C)
    rep = lambda shape: pl.BlockSpec(shape, lambda i: (0,) * len(shape))
    return pl.pallas_call(
        _gnavg_kernel,
        grid=(N_BLOCKS,),
        in_specs=[
            pl.BlockSpec((1, 1, BLOCK_R), lambda i: (i, 0, 0)),
            pl.BlockSpec((BLOCK_R, D_C), lambda i: (i, 0)),
            rep((N_GRAPHS_C, D_C)),
            rep((D_C, D_C)),
            rep((D_C, D_C)),
            rep((1, D_C)),
            rep((D_C, D_C)),
            rep((1, D_C)),
        ],
        out_specs=rep((N_GRAPHS_C, D_C)),
        out_shape=jax.ShapeDtypeStruct((N_GRAPHS_C, D_C), jnp.float32),
        scratch_shapes=[
            pltpu.VMEM((N_GRAPHS_C, D_C), jnp.float32),
            pltpu.VMEM((N_GRAPHS_C, 8), jnp.float32),
        ],
    )(ids, x, u, w1a, w1b, b1r, W2, b2r)


# speedup vs baseline: 1.2166x; 1.0073x over previous
import jax
import jax.numpy as jnp
from jax.experimental import pallas as pl
from jax.experimental.pallas import tpu as pltpu

N_NODES_C = 100000
N_GRAPHS_C = 256
D_C = 128
BLOCK_R = 10000
N_BLOCKS = N_NODES_C // BLOCK_R


def _gnavg_kernel(ids_ref, x_ref, u_ref, w1a_ref, w1b_ref, b1_ref, w2_ref,
                  b2_ref, out_ref, sums_ref, cnts_ref):
    i = pl.program_id(0)

    @pl.when(i == 0)
    def _():
        sums_ref[...] = jnp.zeros_like(sums_ref)
        cnts_ref[...] = jnp.zeros_like(cnts_ref)

    ids = ids_ref[0, :, :].astype(jnp.int16)
    seg = jax.lax.broadcasted_iota(jnp.int16, (N_GRAPHS_C, BLOCK_R), 0)
    onehot = jnp.where(ids == seg, jnp.bfloat16(1.0), jnp.bfloat16(0.0))
    xb = x_ref[...].astype(jnp.bfloat16)
    psum = jax.lax.dot_general(
        onehot, xb, (((1,), (0,)), ((), ())),
        preferred_element_type=jnp.float32)
    sums_ref[...] += psum
    ones = jnp.ones((BLOCK_R, 8), jnp.bfloat16)
    pcnt = jax.lax.dot_general(
        onehot, ones, (((1,), (0,)), ((), ())),
        preferred_element_type=jnp.float32)
    cnts_ref[...] += pcnt

    @pl.when(i == N_BLOCKS - 1)
    def _():
        counts = cnts_ref[:, 0:1]
        mean = sums_ref[...] / jnp.maximum(counts, 1.0)
        u1 = jax.lax.dot_general(
            mean, w1a_ref[...], (((1,), (0,)), ((), ())),
            preferred_element_type=jnp.float32)
        u1 += jax.lax.dot_general(
            u_ref[...], w1b_ref[...], (((1,), (0,)), ((), ())),
            preferred_element_type=jnp.float32)
        u1 = jnp.maximum(u1 + b1_ref[...], 0.0)
        out = jax.lax.dot_general(
            u1, w2_ref[...], (((1,), (0,)), ((), ())),
            preferred_element_type=jnp.float32)
        out_ref[...] = out + b2_ref[...]


@jax.jit
def kernel(x, u, graph_ids, W1, b1, W2, b2):
    ids = graph_ids.astype(jnp.int32).reshape(N_BLOCKS, 1, BLOCK_R)
    w1a = W1[:D_C]
    w1b = W1[D_C:]
    b1r = b1.reshape(1, D_C)
    b2r = b2.reshape(1, D_C)
    rep = lambda shape: pl.BlockSpec(shape, lambda i: (0,) * len(shape))
    return pl.pallas_call(
        _gnavg_kernel,
        grid=(N_BLOCKS,),
        in_specs=[
            pl.BlockSpec((1, 1, BLOCK_R), lambda i: (i, 0, 0)),
            pl.BlockSpec((BLOCK_R, D_C), lambda i: (i, 0)),
            rep((N_GRAPHS_C, D_C)),
            rep((D_C, D_C)),
            rep((D_C, D_C)),
            rep((1, D_C)),
            rep((D_C, D_C)),
            rep((1, D_C)),
        ],
        out_specs=rep((N_GRAPHS_C, D_C)),
        out_shape=jax.ShapeDtypeStruct((N_GRAPHS_C, D_C), jnp.float32),
        scratch_shapes=[
            pltpu.VMEM((N_GRAPHS_C, D_C), jnp.float32),
            pltpu.VMEM((N_GRAPHS_C, 8), jnp.float32),
        ],
    )(ids, x, u, w1a, w1b, b1r, W2, b2r)
